# baseline (speedup 1.0000x reference)
import functools

import jax
import jax.numpy as jnp
from jax import lax
from jax.experimental import pallas as pl
from jax.experimental.pallas import tpu as pltpu
from jax.experimental.pallas import tpu_sc as plsc

_NV = 40962
_NF = 81920
_B = 4
_CIN = 32
_COUT = 32
_D = _B * _CIN
_NVP = 41984
_NVC = 41472
_NW = 32
_CF = 32
_FW = _NF // _NW
_NCA = _FW // _CF
_CV = 16
_VW = _NVP // _NW
_NCB = _VW // _CV
_TV = 512


def _c16(i):
    return jnp.full((16,), i, jnp.int32)


def _serial_chunks(nch, idx_copies, gathers, out_copy, compute):
    def body(ci, carry):
        for c in idx_copies(ci, 0):
            c.start()
        for c in idx_copies(ci, 0):
            c.wait()
        gs = gathers(0)
        for c in gs:
            c.start()
        for c in gs:
            c.wait()
        compute(0)
        oc = out_copy(ci, 0)
        oc.start()
        oc.wait()
        return carry

    lax.fori_loop(0, nch, body, 0)


def _pipeline(nch, idx_copies, gathers, out_copy, compute):
    def start(cs):
        for c in cs:
            c.start()

    def wait(cs):
        for c in cs:
            c.wait()

    def sync(cs):
        start(cs)
        wait(cs)

    sync(idx_copies(0, 0))
    start(gathers(0))
    sync(idx_copies(1, 1))
    wait(gathers(0))
    start(gathers(1))
    compute(0)
    out_copy(0, 0).start()
    start(idx_copies(2, 0))
    wait(gathers(1))
    wait(idx_copies(2, 0))
    start(gathers(0))
    compute(1)
    out_copy(1, 1).start()
    start(idx_copies(3, 1))

    def body(ci, b):
        wait(gathers(b))
        wait(idx_copies(ci + 1, 1 - b))
        start(gathers(1 - b))
        out_copy(ci - 2, b).wait()
        compute(b)
        out_copy(ci, b).start()
        start(idx_copies(ci + 2, b))

    def pairf(cj, carry):
        ci = 2 + cj * 2
        body(ci, 0)
        body(ci + 1, 1)
        return carry

    lax.fori_loop(0, (nch - 4) // 2, pairf, 0)

    ci = nch - 2
    wait(gathers(0))
    wait(idx_copies(ci + 1, 1))
    start(gathers(1))
    out_copy(ci - 2, 0).wait()
    compute(0)
    out_copy(ci, 0).start()
    wait(gathers(1))
    out_copy(ci - 1, 1).wait()
    compute(1)
    out_copy(ci + 1, 1).start()
    out_copy(ci, 0).wait()
    out_copy(ci + 1, 1).wait()


def _mesh():
    return plsc.VectorSubcoreMesh(core_axis_name="c", subcore_axis_name="s")


def _transpose_x(xr):
    def t(x_ref, o_ref):
        o_ref[...] = x_ref[...].T

    return pl.pallas_call(
        t,
        grid=(_NVC // _TV,),
        in_specs=[pl.BlockSpec((_D, _TV), lambda i: (0, i))],
        out_specs=pl.BlockSpec((_TV, _D), lambda i: (i, 0)),
        out_shape=jax.ShapeDtypeStruct((_NVC, _D), jnp.float32),
    )(xr)


def _stage_a(x2, gcols, gvals, ew, ns):
    @functools.partial(
        pl.kernel,
        out_type=jax.ShapeDtypeStruct((_NF, _D), jnp.float32),
        mesh=_mesh(),
        scratch_types=(
            [pltpu.VMEM((_CF * 9,), jnp.int32)] * 2
            + [pltpu.VMEM((_CF * 9,), jnp.float32)] * 2
            + [pltpu.VMEM((_CF * 3,), jnp.float32)] * 4
            + [pltpu.VMEM((_CF * 9, _D), jnp.float32)] * 2
            + [pltpu.VMEM((_CF, _D), jnp.float32)] * 2
            + [pltpu.SemaphoreType.DMA] * 6
        ),
        compiler_params=pltpu.CompilerParams(needs_layout_passes=False),
    )
    def k(x2_hbm, cols_hbm, gv_hbm, ew_hbm, ns_hbm, gf_hbm,
          colsv0, colsv1, gvv0, gvv1, ewv0, ewv1, nsv0, nsv1,
          rowsv0, rowsv1, outv0, outv1,
          isem0, isem1, gsem0, gsem1, osem0, osem1):
        wid = lax.axis_index("c") * 16 + lax.axis_index("s")
        colsv = (colsv0, colsv1)
        gvv = (gvv0, gvv1)
        ewv = (ewv0, ewv1)
        nsv = (nsv0, nsv1)
        rowsv = (rowsv0, rowsv1)
        outv = (outv0, outv1)
        isem = (isem0, isem1)
        gsem = (gsem0, gsem1)
        osem = (osem0, osem1)

        def idx_copies(ci, b):
            base = wid * _FW + ci * _CF
            cps = []
            for j in range(3):
                cps.append(pltpu.make_async_copy(
                    cols_hbm.at[pl.ds(j * 3 * _NF + 3 * base, 3 * _CF)],
                    colsv[b].at[pl.ds(j * 3 * _CF, 3 * _CF)], isem[b]))
                cps.append(pltpu.make_async_copy(
                    gv_hbm.at[pl.ds(j * 3 * _NF + 3 * base, 3 * _CF)],
                    gvv[b].at[pl.ds(j * 3 * _CF, 3 * _CF)], isem[b]))
            cps.append(pltpu.make_async_copy(
                ew_hbm.at[pl.ds(3 * base, 3 * _CF)], ewv[b], isem[b]))
            cps.append(pltpu.make_async_copy(
                ns_hbm.at[pl.ds(3 * base, 3 * _CF)], nsv[b], isem[b]))
            return cps

        def gather_copy(b):
            return pltpu.make_async_copy(x2_hbm.at[colsv[b]], rowsv[b], gsem[b])

        def out_copy(ci, b):
            base = wid * _FW + ci * _CF
            return pltpu.make_async_copy(
                outv[b], gf_hbm.at[pl.ds(base, _CF)], osem[b])

        def compute(b):
            def face(f, c2):
                acc_e = [jnp.zeros((16,), jnp.float32) for _ in range(8)]
                acc_n = [jnp.zeros((16,), jnp.float32) for _ in range(8)]
                f3 = jnp.broadcast_to(f * 3, (16,))
                for j in range(3):
                    ewj = plsc.load_gather(ewv[b], [f3 + _c16(j)])
                    nsj = plsc.load_gather(nsv[b], [f3 + _c16(j)])
                    for t in range(3):
                        gv = plsc.load_gather(gvv[b], [f3 + _c16(j * 3 * _CF + t)])
                        we = gv * ewj
                        wn = gv * nsj
                        r = f * 3 + (j * 3 * _CF + t)
                        for cc in range(8):
                            rv = rowsv[b][r, pl.ds(cc * 16, 16)]
                            acc_e[cc] = acc_e[cc] + we * rv
                            acc_n[cc] = acc_n[cc] + wn * rv
                for i in range(4):
                    outv[b][f, pl.ds(i * 16, 16)] = plsc.bitcast(
                        plsc.pack(acc_e[2 * i], acc_e[2 * i + 1],
                                  format=plsc.PackFormat.INTERLEAVED),
                        jnp.float32)
                    outv[b][f, pl.ds(64 + i * 16, 16)] = plsc.bitcast(
                        plsc.pack(acc_n[2 * i], acc_n[2 * i + 1],
                                  format=plsc.PackFormat.INTERLEAVED),
                        jnp.float32)
                return c2

            lax.fori_loop(0, _CF, face, 0)

        _pipeline(_NCA, idx_copies, lambda b: [gather_copy(b)], out_copy, compute)

    return k(x2, gcols, gvals, ew, ns)


def _stage_b(x2, lc, fc, lv, fv, gf):
    @functools.partial(
        pl.kernel,
        out_type=jax.ShapeDtypeStruct((_NVP, 192), jnp.float32),
        mesh=_mesh(),
        scratch_types=(
            [pltpu.VMEM((_CV * 7,), jnp.int32)] * 2
            + [pltpu.VMEM((_CV * 6,), jnp.int32)] * 2
            + [pltpu.VMEM((_CV * 7,), jnp.float32)] * 2
            + [pltpu.VMEM((_CV * 6,), jnp.float32)] * 2
            + [pltpu.VMEM((_CV * 7, _D), jnp.float32)] * 2
            + [pltpu.VMEM((_CV * 6, _D), jnp.float32)] * 2
            + [pltpu.VMEM((_CV, 192), jnp.float32)] * 2
            + [pltpu.SemaphoreType.DMA] * 6
        ),
        compiler_params=pltpu.CompilerParams(needs_layout_passes=False),
    )
    def k(x2_hbm, lc_hbm, fc_hbm, lv_hbm, fv_hbm, gf_hbm, feat_hbm,
          lcv0, lcv1, fcv0, fcv1, lvv0, lvv1, fvv0, fvv1,
          lrows0, lrows1, grows0, grows1, featv0, featv1,
          isem0, isem1, gsem0, gsem1, osem0, osem1):
        wid = lax.axis_index("c") * 16 + lax.axis_index("s")
        lcv = (lcv0, lcv1)
        fcv = (fcv0, fcv1)
        lvv = (lvv0, lvv1)
        fvv = (fvv0, fvv1)
        lrows = (lrows0, lrows1)
        grows = (grows0, grows1)
        featv = (featv0, featv1)
        isem = (isem0, isem1)
        gsem = (gsem0, gsem1)
        osem = (osem0, osem1)

        def idx_copies(ci, b):
            vb = wid * _VW + ci * _CV
            return [
                pltpu.make_async_copy(lc_hbm.at[pl.ds(vb * 7, _CV * 7)],
                                      lcv[b], isem[b]),
                pltpu.make_async_copy(fc_hbm.at[pl.ds(vb * 6, _CV * 6)],
                                      fcv[b], isem[b]),
                pltpu.make_async_copy(lv_hbm.at[pl.ds(vb * 7, _CV * 7)],
                                      lvv[b], isem[b]),
                pltpu.make_async_copy(fv_hbm.at[pl.ds(vb * 6, _CV * 6)],
                                      fvv[b], isem[b]),
            ]

        def gather_copies(b):
            return [
                pltpu.make_async_copy(x2_hbm.at[lcv[b]], lrows[b], gsem[b]),
                pltpu.make_async_copy(gf_hbm.at[fcv[b]], grows[b], gsem[b]),
            ]

        def out_copy(ci, b):
            vb = wid * _VW + ci * _CV
            return pltpu.make_async_copy(
                featv[b], feat_hbm.at[pl.ds(vb, _CV)], osem[b])

        def compute(b):
            def vert(v, cy):
                v7 = jnp.broadcast_to(v * 7, (16,))
                v6 = jnp.broadcast_to(v * 6, (16,))
                accl = [jnp.zeros((16,), jnp.float32) for _ in range(8)]
                for t in range(7):
                    w = plsc.load_gather(lvv[b], [v7 + _c16(t)])
                    r = v * 7 + t
                    for cc in range(8):
                        accl[cc] = accl[cc] + w * lrows[b][r, pl.ds(cc * 16, 16)]
                for i in range(4):
                    featv[b][v, pl.ds(i * 16, 16)] = plsc.bitcast(
                        plsc.pack(accl[2 * i], accl[2 * i + 1],
                                  format=plsc.PackFormat.INTERLEAVED),
                        jnp.float32)
                acce = [jnp.zeros((16,), jnp.float32) for _ in range(8)]
                accn = [jnp.zeros((16,), jnp.float32) for _ in range(8)]
                for t in range(6):
                    w = plsc.load_gather(fvv[b], [v6 + _c16(t)])
                    r = v * 6 + t
                    for i in range(4):
                        e0, e1 = plsc.unpack(
                            plsc.bitcast(grows[b][r, pl.ds(i * 16, 16)],
                                         jnp.bfloat16),
                            format=plsc.PackFormat.INTERLEAVED)
                        n0, n1 = plsc.unpack(
                            plsc.bitcast(grows[b][r, pl.ds(64 + i * 16, 16)],
                                         jnp.bfloat16),
                            format=plsc.PackFormat.INTERLEAVED)
                        acce[2 * i] = acce[2 * i] + w * e0
                        acce[2 * i + 1] = acce[2 * i + 1] + w * e1
                        accn[2 * i] = accn[2 * i] + w * n0
                        accn[2 * i + 1] = accn[2 * i + 1] + w * n1
                for i in range(4):
                    featv[b][v, pl.ds(64 + i * 16, 16)] = plsc.bitcast(
                        plsc.pack(acce[2 * i], acce[2 * i + 1],
                                  format=plsc.PackFormat.INTERLEAVED),
                        jnp.float32)
                    featv[b][v, pl.ds(128 + i * 16, 16)] = plsc.bitcast(
                        plsc.pack(accn[2 * i], accn[2 * i + 1],
                                  format=plsc.PackFormat.INTERLEAVED),
                        jnp.float32)
                return cy

            lax.fori_loop(0, _CV, vert, 0)

        _pipeline(_NCB, idx_copies, gather_copies, out_copy, compute)

    return k(x2, lc, fc, lv, fv, gf)


def _stage_c(x2, feat, wta, wtb, biasc):
    def mm(f_ref, x_ref, wa_ref, wb_ref, b_ref, o_ref):
        dn = (((1,), (1,)), ((), ()))
        acc = lax.dot_general(wa_ref[...], f_ref[...], dn,
                              preferred_element_type=jnp.float32)
        acc = acc + lax.dot_general(wb_ref[...], x_ref[...], dn,
                                    preferred_element_type=jnp.float32)
        acc = acc + b_ref[:, 0:1]
        o_ref[...] = acc.reshape(_B, _COUT, _TV)

    return pl.pallas_call(
        mm,
        grid=(_NVC // _TV,),
        in_specs=[
            pl.BlockSpec((_TV, 384), lambda i: (i, 0)),
            pl.BlockSpec((_TV, _D), lambda i: (i, 0)),
            pl.BlockSpec((_D, 384), lambda i: (0, 0)),
            pl.BlockSpec((_D, _D), lambda i: (0, 0)),
            pl.BlockSpec((_D, _D), lambda i: (0, 0)),
        ],
        out_specs=pl.BlockSpec((_B, _COUT, _TV), lambda i: (0, 0, i)),
        out_shape=jax.ShapeDtypeStruct((_B, _COUT, _NV), jnp.float32),
    )(feat, x2, wta, wtb, biasc)


def kernel(x, g_rows, g_cols, g_vals, l_rows, l_cols, l_vals,
           f_rows, f_cols, f_vals, EW, NS, coeffs, bias):
    x2p = _transpose_x(x.reshape(_D, _NV))

    gcols = g_cols.astype(jnp.int32)
    ew_flat = EW.reshape(-1)
    ns_flat = NS.reshape(-1)

    pad_v = _NVP - _NV
    lc = jnp.pad(l_cols.astype(jnp.int32), (0, pad_v * 7))
    fc = jnp.pad(f_cols.astype(jnp.int32), (0, pad_v * 6))
    lv = jnp.pad(l_vals, (0, pad_v * 7))
    fv = jnp.pad(f_vals, (0, pad_v * 6))

    ct = jnp.transpose(coeffs, (2, 1, 0))
    eye_b = jnp.eye(_B, dtype=jnp.float32)
    w5 = ct[:, None, :, None, :] * eye_b[None, :, None, :, None]
    wbig_t = w5.reshape(4 * _D, _D).T
    wtb = wbig_t[:, 0:_D]
    p32 = jnp.arange(32)
    sigma32 = jnp.where(p32 % 2 == 0, p32 // 2, 16 + p32 // 2)
    sigma = (jnp.arange(384) // 32) * 32 + sigma32[jnp.arange(384) % 32]
    wta = wbig_t[:, _D:][:, sigma].astype(jnp.bfloat16)
    biasc = jnp.broadcast_to(jnp.tile(bias, _B)[:, None], (_D, _D))

    gf = _stage_a(x2p, gcols, g_vals, ew_flat, ns_flat)
    feat = _stage_b(x2p, lc, fc, lv, fv, gf)
    feat_bf = lax.bitcast_convert_type(feat, jnp.bfloat16).reshape(_NVP, 384)
    return _stage_c(x2p, feat_bf, wta, wtb, biasc)


# speedup vs baseline: 1.4556x; 1.4556x over previous
import functools

import jax
import jax.numpy as jnp
from jax import lax
from jax.experimental import pallas as pl
from jax.experimental.pallas import tpu as pltpu
from jax.experimental.pallas import tpu_sc as plsc

_NV = 40962
_NF = 81920
_B = 4
_CIN = 32
_COUT = 32
_D = _B * _CIN
_NVP = 41984
_NVC = 41472
_NW = 32
_CF = 32
_FW = _NF // _NW
_NCA = _FW // _CF
_CV = 16
_VW = _NVP // _NW
_NCB = _VW // _CV
_TV = 512


def _c16(i):
    return jnp.full((16,), i, jnp.int32)


def _serial_chunks(nch, idx_copies, gathers, out_copy, compute):
    def body(ci, carry):
        for c in idx_copies(ci, 0):
            c.start()
        for c in idx_copies(ci, 0):
            c.wait()
        gs = gathers(0)
        for c in gs:
            c.start()
        for c in gs:
            c.wait()
        compute(0)
        oc = out_copy(ci, 0)
        oc.start()
        oc.wait()
        return carry

    lax.fori_loop(0, nch, body, 0)


def _pipeline(nch, idx_copies, gathers, out_copy, compute):
    def start(cs):
        for c in cs:
            c.start()

    def wait(cs):
        for c in cs:
            c.wait()

    def sync(cs):
        start(cs)
        wait(cs)

    sync(idx_copies(0, 0))
    start(gathers(0))
    sync(idx_copies(1, 1))
    wait(gathers(0))
    start(gathers(1))
    compute(0)
    out_copy(0, 0).start()
    start(idx_copies(2, 0))
    wait(gathers(1))
    wait(idx_copies(2, 0))
    start(gathers(0))
    compute(1)
    out_copy(1, 1).start()
    start(idx_copies(3, 1))

    def body(ci, b):
        wait(gathers(b))
        wait(idx_copies(ci + 1, 1 - b))
        start(gathers(1 - b))
        out_copy(ci - 2, b).wait()
        compute(b)
        out_copy(ci, b).start()
        start(idx_copies(ci + 2, b))

    def pairf(cj, carry):
        ci = 2 + cj * 2
        body(ci, 0)
        body(ci + 1, 1)
        return carry

    lax.fori_loop(0, (nch - 4) // 2, pairf, 0)

    ci = nch - 2
    wait(gathers(0))
    wait(idx_copies(ci + 1, 1))
    start(gathers(1))
    out_copy(ci - 2, 0).wait()
    compute(0)
    out_copy(ci, 0).start()
    wait(gathers(1))
    out_copy(ci - 1, 1).wait()
    compute(1)
    out_copy(ci + 1, 1).start()
    out_copy(ci, 0).wait()
    out_copy(ci + 1, 1).wait()


def _mesh():
    return plsc.VectorSubcoreMesh(core_axis_name="c", subcore_axis_name="s")


def _transpose_x(xr):
    def t(x_ref, o_ref):
        o_ref[...] = x_ref[...].T

    return pl.pallas_call(
        t,
        grid=(_NVC // _TV,),
        in_specs=[pl.BlockSpec((_D, _TV), lambda i: (0, i))],
        out_specs=pl.BlockSpec((_TV, _D), lambda i: (i, 0)),
        out_shape=jax.ShapeDtypeStruct((_NVC, _D), jnp.float32),
    )(xr)


def _stage_a(x2, gcols, gvals, ew, ns):
    @functools.partial(
        pl.kernel,
        out_type=jax.ShapeDtypeStruct((_NF, _D), jnp.float32),
        mesh=_mesh(),
        scratch_types=(
            [pltpu.VMEM((_CF * 9,), jnp.int32)] * 2
            + [pltpu.VMEM((_CF * 9,), jnp.float32)] * 2
            + [pltpu.VMEM((_CF * 3,), jnp.float32)] * 4
            + [pltpu.VMEM((_CF * 9, _D), jnp.float32)] * 2
            + [pltpu.VMEM((_CF, _D), jnp.float32)] * 2
            + [pltpu.SemaphoreType.DMA] * 6
        ),
        compiler_params=pltpu.CompilerParams(needs_layout_passes=False),
    )
    def k(x2_hbm, cols_hbm, gv_hbm, ew_hbm, ns_hbm, gf_hbm,
          colsv0, colsv1, gvv0, gvv1, ewv0, ewv1, nsv0, nsv1,
          rowsv0, rowsv1, outv0, outv1,
          isem0, isem1, gsem0, gsem1, osem0, osem1):
        wid = lax.axis_index("c") * 16 + lax.axis_index("s")
        colsv = (colsv0, colsv1)
        gvv = (gvv0, gvv1)
        ewv = (ewv0, ewv1)
        nsv = (nsv0, nsv1)
        rowsv = (rowsv0, rowsv1)
        outv = (outv0, outv1)
        isem = (isem0, isem1)
        gsem = (gsem0, gsem1)
        osem = (osem0, osem1)

        def idx_copies(ci, b):
            base = wid * _FW + ci * _CF
            cps = []
            for j in range(3):
                cps.append(pltpu.make_async_copy(
                    cols_hbm.at[pl.ds(j * 3 * _NF + 3 * base, 3 * _CF)],
                    colsv[b].at[pl.ds(j * 3 * _CF, 3 * _CF)], isem[b]))
                cps.append(pltpu.make_async_copy(
                    gv_hbm.at[pl.ds(j * 3 * _NF + 3 * base, 3 * _CF)],
                    gvv[b].at[pl.ds(j * 3 * _CF, 3 * _CF)], isem[b]))
            cps.append(pltpu.make_async_copy(
                ew_hbm.at[pl.ds(3 * base, 3 * _CF)], ewv[b], isem[b]))
            cps.append(pltpu.make_async_copy(
                ns_hbm.at[pl.ds(3 * base, 3 * _CF)], nsv[b], isem[b]))
            return cps

        def gather_copy(b):
            return pltpu.make_async_copy(x2_hbm.at[colsv[b]], rowsv[b], gsem[b])

        def out_copy(ci, b):
            base = wid * _FW + ci * _CF
            return pltpu.make_async_copy(
                outv[b], gf_hbm.at[pl.ds(base, _CF)], osem[b])

        def compute(b):
            def face(f, c2):
                acc_e = [jnp.zeros((16,), jnp.float32) for _ in range(8)]
                acc_n = [jnp.zeros((16,), jnp.float32) for _ in range(8)]
                f3 = jnp.broadcast_to(f * 3, (16,))
                for j in range(3):
                    ewj = plsc.load_gather(ewv[b], [f3 + _c16(j)])
                    nsj = plsc.load_gather(nsv[b], [f3 + _c16(j)])
                    for t in range(3):
                        gv = plsc.load_gather(gvv[b], [f3 + _c16(j * 3 * _CF + t)])
                        we = gv * ewj
                        wn = gv * nsj
                        r = f * 3 + (j * 3 * _CF + t)
                        for cc in range(8):
                            rv = rowsv[b][r, pl.ds(cc * 16, 16)]
                            acc_e[cc] = acc_e[cc] + we * rv
                            acc_n[cc] = acc_n[cc] + wn * rv
                for i in range(4):
                    outv[b][f, pl.ds(i * 16, 16)] = plsc.bitcast(
                        plsc.pack(acc_e[2 * i], acc_e[2 * i + 1],
                                  format=plsc.PackFormat.INTERLEAVED),
                        jnp.float32)
                    outv[b][f, pl.ds(64 + i * 16, 16)] = plsc.bitcast(
                        plsc.pack(acc_n[2 * i], acc_n[2 * i + 1],
                                  format=plsc.PackFormat.INTERLEAVED),
                        jnp.float32)
                return c2

            lax.fori_loop(0, _CF, face, 0)

        _pipeline(_NCA, idx_copies, lambda b: [gather_copy(b)], out_copy, compute)

    return k(x2, gcols, gvals, ew, ns)


def _stage_b(x2, lc, fc, lv, fv, gf):
    @functools.partial(
        pl.kernel,
        out_type=jax.ShapeDtypeStruct((_NVP, 384), jnp.float32),
        mesh=_mesh(),
        scratch_types=(
            [pltpu.VMEM((_CV * 7,), jnp.int32)] * 2
            + [pltpu.VMEM((_CV * 6,), jnp.int32)] * 2
            + [pltpu.VMEM((_CV * 7,), jnp.float32)] * 2
            + [pltpu.VMEM((_CV * 6,), jnp.float32)] * 2
            + [pltpu.VMEM((_CV * 7, _D), jnp.float32)] * 2
            + [pltpu.VMEM((_CV * 6, _D), jnp.float32)] * 2
            + [pltpu.VMEM((_CV, 384), jnp.float32)] * 2
            + [pltpu.SemaphoreType.DMA] * 6
        ),
        compiler_params=pltpu.CompilerParams(needs_layout_passes=False),
    )
    def k(x2_hbm, lc_hbm, fc_hbm, lv_hbm, fv_hbm, gf_hbm, feat_hbm,
          lcv0, lcv1, fcv0, fcv1, lvv0, lvv1, fvv0, fvv1,
          lrows0, lrows1, grows0, grows1, featv0, featv1,
          isem0, isem1, gsem0, gsem1, osem0, osem1):
        wid = lax.axis_index("c") * 16 + lax.axis_index("s")
        lcv = (lcv0, lcv1)
        fcv = (fcv0, fcv1)
        lvv = (lvv0, lvv1)
        fvv = (fvv0, fvv1)
        lrows = (lrows0, lrows1)
        grows = (grows0, grows1)
        featv = (featv0, featv1)
        isem = (isem0, isem1)
        gsem = (gsem0, gsem1)
        osem = (osem0, osem1)

        def idx_copies(ci, b):
            vb = wid * _VW + ci * _CV
            return [
                pltpu.make_async_copy(lc_hbm.at[pl.ds(vb * 7, _CV * 7)],
                                      lcv[b], isem[b]),
                pltpu.make_async_copy(fc_hbm.at[pl.ds(vb * 6, _CV * 6)],
                                      fcv[b], isem[b]),
                pltpu.make_async_copy(lv_hbm.at[pl.ds(vb * 7, _CV * 7)],
                                      lvv[b], isem[b]),
                pltpu.make_async_copy(fv_hbm.at[pl.ds(vb * 6, _CV * 6)],
                                      fvv[b], isem[b]),
            ]

        def gather_copies(b):
            return [
                pltpu.make_async_copy(x2_hbm.at[lcv[b]], lrows[b], gsem[b]),
                pltpu.make_async_copy(gf_hbm.at[fcv[b]], grows[b], gsem[b]),
            ]

        def out_copy(ci, b):
            vb = wid * _VW + ci * _CV
            return pltpu.make_async_copy(
                featv[b], feat_hbm.at[pl.ds(vb, _CV)], osem[b])

        def compute(b):
            def vert(v, cy):
                v7 = jnp.broadcast_to(v * 7, (16,))
                v6 = jnp.broadcast_to(v * 6, (16,))
                accl = [jnp.zeros((16,), jnp.float32) for _ in range(8)]
                for t in range(7):
                    w = plsc.load_gather(lvv[b], [v7 + _c16(t)])
                    r = v * 7 + t
                    for cc in range(8):
                        accl[cc] = accl[cc] + w * lrows[b][r, pl.ds(cc * 16, 16)]
                for cc in range(8):
                    featv[b][v, pl.ds(cc * 16, 16)] = accl[cc]
                acce = [jnp.zeros((16,), jnp.float32) for _ in range(8)]
                accn = [jnp.zeros((16,), jnp.float32) for _ in range(8)]
                for t in range(6):
                    w = plsc.load_gather(fvv[b], [v6 + _c16(t)])
                    r = v * 6 + t
                    for i in range(4):
                        e0, e1 = plsc.unpack(
                            plsc.bitcast(grows[b][r, pl.ds(i * 16, 16)],
                                         jnp.bfloat16),
                            format=plsc.PackFormat.INTERLEAVED)
                        n0, n1 = plsc.unpack(
                            plsc.bitcast(grows[b][r, pl.ds(64 + i * 16, 16)],
                                         jnp.bfloat16),
                            format=plsc.PackFormat.INTERLEAVED)
                        acce[2 * i] = acce[2 * i] + w * e0
                        acce[2 * i + 1] = acce[2 * i + 1] + w * e1
                        accn[2 * i] = accn[2 * i] + w * n0
                        accn[2 * i + 1] = accn[2 * i + 1] + w * n1
                for cc in range(8):
                    featv[b][v, pl.ds(128 + cc * 16, 16)] = acce[cc]
                    featv[b][v, pl.ds(256 + cc * 16, 16)] = accn[cc]
                return cy

            lax.fori_loop(0, _CV, vert, 0)

        _pipeline(_NCB, idx_copies, gather_copies, out_copy, compute)

    return k(x2, lc, fc, lv, fv, gf)


def _stage_c(x2, feat, wta, wtb, biasc):
    def mm(f_ref, x_ref, wa_ref, wb_ref, b_ref, o_ref):
        dn = (((1,), (1,)), ((), ()))
        acc = lax.dot_general(wa_ref[...], f_ref[...], dn,
                              preferred_element_type=jnp.float32)
        acc = acc + lax.dot_general(wb_ref[...], x_ref[...], dn,
                                    preferred_element_type=jnp.float32)
        acc = acc + b_ref[:, 0:1]
        o_ref[...] = acc.reshape(_B, _COUT, _TV)

    return pl.pallas_call(
        mm,
        grid=(_NVC // _TV,),
        in_specs=[
            pl.BlockSpec((_TV, 384), lambda i: (i, 0)),
            pl.BlockSpec((_TV, _D), lambda i: (i, 0)),
            pl.BlockSpec((_D, 384), lambda i: (0, 0)),
            pl.BlockSpec((_D, _D), lambda i: (0, 0)),
            pl.BlockSpec((_D, _D), lambda i: (0, 0)),
        ],
        out_specs=pl.BlockSpec((_B, _COUT, _TV), lambda i: (0, 0, i)),
        out_shape=jax.ShapeDtypeStruct((_B, _COUT, _NV), jnp.float32),
    )(feat, x2, wta, wtb, biasc)


def kernel(x, g_rows, g_cols, g_vals, l_rows, l_cols, l_vals,
           f_rows, f_cols, f_vals, EW, NS, coeffs, bias):
    x2p = _transpose_x(x.reshape(_D, _NV))

    gcols = g_cols.astype(jnp.int32)
    ew_flat = EW.reshape(-1)
    ns_flat = NS.reshape(-1)

    pad_v = _NVP - _NV
    lc = jnp.pad(l_cols.astype(jnp.int32), (0, pad_v * 7))
    fc = jnp.pad(f_cols.astype(jnp.int32), (0, pad_v * 6))
    lv = jnp.pad(l_vals, (0, pad_v * 7))
    fv = jnp.pad(f_vals, (0, pad_v * 6))

    ct = jnp.transpose(coeffs, (2, 1, 0))
    eye_b = jnp.eye(_B, dtype=jnp.float32)
    w5 = ct[:, None, :, None, :] * eye_b[None, :, None, :, None]
    wbig_t = w5.reshape(4 * _D, _D).T
    wtb = wbig_t[:, 0:_D]
    wta = wbig_t[:, _D:]
    biasc = jnp.broadcast_to(jnp.tile(bias, _B)[:, None], (_D, _D))

    gf = _stage_a(x2p, gcols, g_vals, ew_flat, ns_flat)
    feat = _stage_b(x2p, lc, fc, lv, fv, gf)
    return _stage_c(x2p, feat, wta, wtb, biasc)
